# Initial kernel scaffold; baseline (speedup 1.0000x reference)
import jax
import jax.numpy as jnp
from jax.experimental import pallas as pl


def kernel(t, u, w1t, b1, w2t, b2, w3_row, b3, params_vec):
    raise NotImplementedError("write your pallas kernel here")


# speedup vs baseline: 1.6743x; 1.6743x over previous
import jax
import jax.numpy as jnp
from jax import lax
from jax.experimental import pallas as pl
from jax.experimental.pallas import tpu as pltpu


def _rk4_kernel(dtf_ref, params_ref, u0_ref, w1c_ref, b1c_ref, w2T_ref,
                b2c_ref, w3c_ref, b3_ref, out_ref, u_state):
    chunk = pl.program_id(0)
    T = out_ref.shape[0]
    Nx = out_ref.shape[2]

    D0 = params_ref[0]
    D1 = params_ref[1]
    bc_c = params_ref[2]
    bc_ct = params_ref[3]
    s0 = params_ref[4]
    s1 = params_ref[5]
    p_scale = params_ref[6]
    dx = params_ref[7]

    @pl.when(chunk == 0)
    def _():
        u_state[...] = u0_ref[...]

    w1 = w1c_ref[...]
    b1 = b1c_ref[...]
    w2 = w2T_ref[...]
    b2 = b2c_ref[...]
    w3 = w3c_ref[...]
    b3 = b3_ref[0, 0]
    lane = lax.broadcasted_iota(jnp.int32, (1, Nx), 1)
    is_first = lane == 0
    is_last = lane == Nx - 1

    def stage(c):
        h1 = jnp.tanh(w1 * c + b1)
        h2 = jnp.tanh(
            lax.dot(w2, h1, preferred_element_type=jnp.float32) + b2)
        o = jnp.sum(h2 * w3, axis=0, keepdims=True) + b3
        ret = jax.nn.sigmoid(o) * p_scale

        c_prev = pltpu.roll(c, shift=1, axis=1)
        c_next = pltpu.roll(c, shift=Nx - 1, axis=1)
        left_c = jnp.where(is_first, bc_c, c_prev)
        left_ct = jnp.where(is_first, bc_ct, c_prev)
        right = jnp.where(is_last, D0 * dx * (c_prev - c), c_next)

        dret = D0 * ret
        kc = (dret * (s0 * c + s1 * left_c)
              + dret * (s0 * c + s1 * right))
        f = (D1 * (s0 * c + s1 * left_ct)
             + D1 * (s0 * c + s1 * right))
        return kc, f

    def body(i, carry):
        c, ct = carry
        step = chunk * T + i
        dt = dtf_ref[0, step]
        dt_half = dtf_ref[1, step]
        dt_sixth = dtf_ref[2, step]
        k1, f1 = stage(c)
        k2, f2 = stage(c + dt_half * k1)
        k3, f3 = stage(c + dt_half * k2)
        k4, f4 = stage(c + dt * k3)
        c_n = c + dt_sixth * (k1 + 2.0 * k2 + 2.0 * k3 + k4)
        ct_n = ct + dt_sixth * (f1 + 2.0 * f2 + 2.0 * f3 + f4)
        out_ref[i] = jnp.concatenate([c_n, ct_n], axis=0)
        return (c_n, ct_n)

    c_f, ct_f = lax.fori_loop(0, T, body,
                              (u_state[0:1, :], u_state[1:2, :]),
                              unroll=False)
    u_state[0:1, :] = c_f
    u_state[1:2, :] = ct_f


def kernel(t, u, w1t, b1, w2t, b2, w3_row, b3, params_vec,
           chunk_steps=128):
    Nx = u.shape[1]
    H = w1t.shape[1]

    u0 = u[0].astype(jnp.float32)
    t = jnp.asarray(t, jnp.float32)
    dts = t[1:] - t[:-1]
    num_steps = int(dts.shape[0])
    if num_steps == 0:
        return u0[None]

    T = min(int(chunk_steps), num_steps)
    num_chunks = -(-num_steps // T)
    padded = num_chunks * T
    if padded > num_steps:
        dts = jnp.concatenate(
            [dts, jnp.zeros((padded - num_steps,), jnp.float32)])
    dt_facs = jnp.stack([dts, 0.5 * dts, dts / 6.0], axis=0)

    u0T = u0.T
    w1c = w1t.T
    b1c = b1.T
    w2T = w2t.T
    b2c = b2.T
    w3c = w3_row.T

    const = lambda c, *_: (0, 0)

    traj = pl.pallas_call(
        _rk4_kernel,
        out_shape=jax.ShapeDtypeStruct((padded, 2, Nx), jnp.float32),
        grid_spec=pltpu.PrefetchScalarGridSpec(
            num_scalar_prefetch=2,
            grid=(num_chunks,),
            in_specs=[
                pl.BlockSpec((2, Nx), const),
                pl.BlockSpec((H, 1), const),
                pl.BlockSpec((H, 1), const),
                pl.BlockSpec((H, H), const),
                pl.BlockSpec((H, 1), const),
                pl.BlockSpec((H, 1), const),
                pl.BlockSpec((1, 1), const),
            ],
            out_specs=pl.BlockSpec((T, 2, Nx), lambda c, *_: (c, 0, 0)),
            scratch_shapes=[pltpu.VMEM((2, Nx), jnp.float32)],
        ),
        compiler_params=pltpu.CompilerParams(
            dimension_semantics=("arbitrary",)),
    )(dt_facs, params_vec, u0T, w1c, b1c, w2T, b2c, w3c, b3)

    out = jnp.transpose(traj[:num_steps], (0, 2, 1))
    return jnp.concatenate([u0[None], out], axis=0)
